# baseline (speedup 1.0000x reference)
import functools

import jax
import jax.numpy as jnp
from jax import lax
from jax.experimental import pallas as pl
from jax.experimental.pallas import tpu as pltpu
from jax.experimental.pallas import tpu_sc as plsc

_START = 40000
_END = 100000
_DENS = (0.5, 0.5, 1.0)
_NROWS = 6144
_NCOLS = 1024
_BR = 8
_BC = 4
_SR = 256
_SC = 256
_CHUNK = 1024
_NCHUNK = _NROWS // _CHUNK
_CPG = _NCHUNK // 3
_SRC = _CHUNK // _BR

_NV = 65536
_VPW = _NV // 16
_HPAD = 2304
_SHIFTS = (20, 9, 0)
_MASKS = (0x7FF, 0x7FF, 0x1FF)


def _scores_body(w_ref, s_ref):
    x = w_ref[...]
    y = jnp.sum((x * x).reshape(_SRC, _BR, _NCOLS), axis=1)
    yt = y.T
    st = jnp.sum(yt.reshape(_SC, _BC, _SRC), axis=1)
    s_ref[...] = st.T


def _expand_body(s_ref, ts_ref, o_ref):
    c = pl.program_id(0)
    t = ts_ref[16 * (c // _CPG)]
    s = s_ref[pl.ds(c * _SRC, _SRC), :]
    m = (jax.lax.bitcast_convert_type(s, jnp.int32) >= t
         ).astype(jnp.float32)
    mt = m.T
    m4 = jnp.broadcast_to(mt[:, None, :], (_SC, _BC, _SRC))
    m4 = m4.reshape(_NCOLS, _SRC)
    m4t = m4.T
    mr = jnp.broadcast_to(m4t[:, None, :], (_SRC, _BR, _NCOLS))
    o_ref[...] = mr.reshape(_CHUNK, _NCOLS)


def _iota16():
    return lax.broadcasted_iota(jnp.int32, (16,), 0)


def _splat(x):
    return jnp.full((16,), x, jnp.int32)


def _sc_select(scores_biased, idx_rep):
    mesh = plsc.VectorSubcoreMesh(core_axis_name="c", subcore_axis_name="s")

    @functools.partial(
        pl.kernel,
        out_type=jax.ShapeDtypeStruct((48,), jnp.int32),
        mesh=mesh,
        scratch_types=[
            pltpu.VMEM((2, _VPW), jnp.int32),
            pltpu.VMEM((48,), jnp.int32),
            pltpu.VMEM((768,), jnp.int32),
            pltpu.VMEM((16,), jnp.int32),
            pltpu.VMEM((16,), jnp.int32),
            pltpu.VMEM((48,), jnp.int32),
            pltpu.VMEM((2, 16), jnp.int32),
            pltpu.VMEM_SHARED((768,), jnp.int32),
            pltpu.VMEM_SHARED((768,), jnp.int32),
            pltpu.VMEM_SHARED((16,), jnp.int32),
            pltpu.VMEM_SHARED((16,), jnp.int32),
        ],
    )
    def k(scores_hbm, idx_hbm, out_hbm, vals, rbuf, pbuf, sbuf, swbuf,
          tbuf, kbuf, part0, part1, st0, st1):

        def lane_sum_splat(acc):
            rbuf[pl.ds(32, 16)] = acc * 0
            for sh in (8, 4, 2, 1):
                rbuf[pl.ds(16, 16)] = acc
                acc = acc + rbuf[pl.ds(16 + sh, 16)]
            for j in range(16):
                rbuf[pl.ds(j, 16)] = acc
            return rbuf[pl.ds(0, 16)]
        c = lax.axis_index("c")
        s = lax.axis_index("s")
        zeros = _splat(0)
        biasv = jnp.full((16,), -(2 ** 31), jnp.int32)

        for gi in range(2):
            gate = c * 2 + gi

            @pl.when(gate < 3)
            def _load():
                base = gate * _NV + s * _VPW
                pltpu.sync_copy(scores_hbm.at[pl.ds(base, _VPW)],
                                vals.at[gi])
                pltpu.sync_copy(idx_hbm.at[pl.ds(gate * 16, 16)],
                                kbuf.at[gi])

        for rnd in range(16):
            b_lo = 30 if rnd == 0 else 28 - 2 * (rnd - 1)
            ncand = 1 if rnd == 0 else 3
            q = 1 << b_lo

            for gi in range(2):
                part = part0 if gi == 0 else part1
                st = st0 if gi == 0 else st1
                gate = c * 2 + gi

                @pl.when(gate < 3)
                def _count():
                    if rnd == 0:
                        xb = biasv
                    else:
                        pltpu.sync_copy(st, sbuf)
                        xb = sbuf[...]
                    cands = [xb + (t + 1) * q for t in range(ncand)]

                    def cstep(i, acc):
                        u = vals[gi, pl.ds(i * 16, 16)]
                        return tuple(
                            a + jnp.where(u < cv, 1, 0)
                            for a, cv in zip(acc, cands))

                    acc = lax.fori_loop(0, _VPW // 16, cstep,
                                        tuple(zeros for _ in range(ncand)),
                                        unroll=8)
                    for t in range(ncand):
                        tbuf[pl.ds(t * 16, 16)] = lane_sum_splat(acc[t])
                    pltpu.sync_copy(
                        tbuf.at[pl.ds(0, 16 * ncand)],
                        part.at[pl.ds(s * 16 * ncand, 16 * ncand)])

            plsc.subcore_barrier()

            for gi in range(2):
                part = part0 if gi == 0 else part1
                st = st0 if gi == 0 else st1
                gate = c * 2 + gi

                @pl.when((gate < 3) & (s == gi))
                def _update():
                    if rnd == 0:
                        xb = biasv
                    else:
                        pltpu.sync_copy(st, sbuf)
                        xb = sbuf[...]
                    kv = kbuf[gi, pl.ds(0, 16)]
                    pltpu.sync_copy(part, pbuf)
                    d = zeros
                    for t in range(ncand):
                        tot = zeros
                        for w in range(16):
                            tot = tot + pbuf[pl.ds(w * 16 * ncand + t * 16,
                                                   16)]
                        d = d + jnp.where(tot <= kv, 1, 0)
                    newx = xb + d * q
                    if rnd < 15:
                        swbuf[...] = newx
                        pltpu.sync_copy(swbuf, st)
                    else:
                        swbuf[...] = newx ^ jnp.full((16,), -(2 ** 31),
                                                     jnp.int32)
                        pltpu.sync_copy(swbuf,
                                        out_hbm.at[pl.ds(gate * 16, 16)])

            plsc.subcore_barrier()

    return k(scores_biased, idx_rep)


def kernel(weight, steps):
    dens = []
    for k in range(3):
        r = 1.0 - (steps - _START) / (_END - _START)
        dens.append(jnp.where(steps < _END,
                              1.0 - (1.0 - _DENS[k]) * (1.0 - r ** 3),
                              _DENS[k]))
    nblk = _SR * _SC
    idx = jnp.stack([jnp.round(nblk * (1.0 - d)).astype(jnp.int32)
                     for d in dens])

    scores = pl.pallas_call(
        _scores_body,
        grid=(_NCHUNK,),
        in_specs=[pl.BlockSpec((_CHUNK, _NCOLS), lambda i: (i, 0))],
        out_specs=pl.BlockSpec((_SRC, _SC), lambda i: (i, 0)),
        out_shape=jax.ShapeDtypeStruct((3 * _SR, _SC), jnp.float32),
    )(weight)

    sbits = jax.lax.bitcast_convert_type(scores.reshape(-1), jnp.int32)
    thresh = _sc_select(sbits ^ jnp.int32(-(2 ** 31)), jnp.repeat(idx, 16))

    out = pl.pallas_call(
        _expand_body,
        grid=(_NCHUNK,),
        in_specs=[
            pl.BlockSpec(memory_space=pltpu.VMEM),
            pl.BlockSpec(memory_space=pltpu.SMEM),
        ],
        out_specs=pl.BlockSpec((_CHUNK, _NCOLS), lambda i: (i, 0)),
        out_shape=jax.ShapeDtypeStruct((_NROWS, _NCOLS), jnp.float32),
    )(scores, thresh)
    return out


# speedup vs baseline: 1.0008x; 1.0008x over previous
import functools

import jax
import jax.numpy as jnp
from jax import lax
from jax.experimental import pallas as pl
from jax.experimental.pallas import tpu as pltpu
from jax.experimental.pallas import tpu_sc as plsc

_START = 40000
_END = 100000
_DENS = (0.5, 0.5, 1.0)
_NROWS = 6144
_NCOLS = 1024
_BR = 8
_BC = 4
_SR = 256
_SC = 256
_CHUNK = 1024
_NCHUNK = _NROWS // _CHUNK
_CPG = _NCHUNK // 3
_SRC = _CHUNK // _BR

_NV = 65536
_VPW = _NV // 16


def _scores_body(w_ref, s_ref):
    x = w_ref[...]
    y = jnp.sum((x * x).reshape(_SRC, _BR, _NCOLS), axis=1)
    yt = y.T
    st = jnp.sum(yt.reshape(_SC, _BC, _SRC), axis=1)
    s_ref[...] = st.T


def _expand_body(s_ref, ts_ref, o_ref):
    c = pl.program_id(0)
    t = ts_ref[16 * (c // _CPG)]
    s = s_ref[pl.ds(c * _SRC, _SRC), :]
    m = (jax.lax.bitcast_convert_type(s, jnp.int32) >= t
         ).astype(jnp.float32)
    mt = m.T
    m4 = jnp.broadcast_to(mt[:, None, :], (_SC, _BC, _SRC))
    m4 = m4.reshape(_NCOLS, _SRC)
    m4t = m4.T
    mr = jnp.broadcast_to(m4t[:, None, :], (_SRC, _BR, _NCOLS))
    o_ref[...] = mr.reshape(_CHUNK, _NCOLS)


def _splat(x):
    return jnp.full((16,), x, jnp.int32)


def _sc_select(scores_biased, idx_rep):
    mesh = plsc.VectorSubcoreMesh(core_axis_name="c", subcore_axis_name="s")

    @functools.partial(
        pl.kernel,
        out_type=jax.ShapeDtypeStruct((48,), jnp.int32),
        mesh=mesh,
        scratch_types=[
            pltpu.VMEM((2, _VPW), jnp.int32),
            pltpu.VMEM((48,), jnp.int32),
            pltpu.VMEM((768,), jnp.int32),
            pltpu.VMEM((16,), jnp.int32),
            pltpu.VMEM((16,), jnp.int32),
            pltpu.VMEM((48,), jnp.int32),
            pltpu.VMEM((2, 16), jnp.int32),
            pltpu.VMEM_SHARED((768,), jnp.int32),
            pltpu.VMEM_SHARED((768,), jnp.int32),
            pltpu.VMEM_SHARED((16,), jnp.int32),
            pltpu.VMEM_SHARED((16,), jnp.int32),
        ],
    )
    def k(scores_hbm, idx_hbm, out_hbm, vals, rbuf, pbuf, sbuf, swbuf,
          tbuf, kbuf, part0, part1, st0, st1):

        def lane_sum_splat(acc):
            rbuf[pl.ds(32, 16)] = acc * 0
            for sh in (8, 4, 2, 1):
                rbuf[pl.ds(16, 16)] = acc
                acc = acc + rbuf[pl.ds(16 + sh, 16)]
            for j in range(16):
                rbuf[pl.ds(j, 16)] = acc
            return rbuf[pl.ds(0, 16)]
        c = lax.axis_index("c")
        s = lax.axis_index("s")
        zeros = _splat(0)
        biasv = jnp.full((16,), -(2 ** 31), jnp.int32)

        for gi in range(2):
            gate = c * 2 + gi

            @pl.when(gate < 3)
            def _load():
                base = gate * _NV + s * _VPW
                pltpu.sync_copy(scores_hbm.at[pl.ds(base, _VPW)],
                                vals.at[gi])
                pltpu.sync_copy(idx_hbm.at[pl.ds(gate * 16, 16)],
                                kbuf.at[gi])

        for rnd in range(16):
            b_lo = 30 if rnd == 0 else 28 - 2 * (rnd - 1)
            ncand = 1 if rnd == 0 else 3
            q = 1 << b_lo

            for gi in range(2):
                part = part0 if gi == 0 else part1
                st = st0 if gi == 0 else st1
                gate = c * 2 + gi

                @pl.when(gate < 3)
                def _count():
                    if rnd == 0:
                        xb = biasv
                    else:
                        pltpu.sync_copy(st, sbuf)
                        xb = sbuf[...]
                    cands = [xb + (t + 1) * q for t in range(ncand)]

                    def cstep(i, acc):
                        u = vals[gi, pl.ds(i * 16, 16)]
                        return tuple(
                            a + jnp.where(u < cv, 1, 0)
                            for a, cv in zip(acc, cands))

                    acc = lax.fori_loop(0, _VPW // 16, cstep,
                                        tuple(zeros for _ in range(ncand)),
                                        unroll=8)
                    for t in range(ncand):
                        tbuf[pl.ds(t * 16, 16)] = lane_sum_splat(acc[t])
                    pltpu.sync_copy(
                        tbuf.at[pl.ds(0, 16 * ncand)],
                        part.at[pl.ds(s * 16 * ncand, 16 * ncand)])

            plsc.subcore_barrier()

            for gi in range(2):
                part = part0 if gi == 0 else part1
                st = st0 if gi == 0 else st1
                gate = c * 2 + gi

                @pl.when((gate < 3) & (s == gi))
                def _update():
                    if rnd == 0:
                        xb = biasv
                    else:
                        pltpu.sync_copy(st, sbuf)
                        xb = sbuf[...]
                    kv = kbuf[gi, pl.ds(0, 16)]
                    pltpu.sync_copy(part, pbuf)
                    d = zeros
                    for t in range(ncand):
                        tot = zeros
                        for w in range(16):
                            tot = tot + pbuf[pl.ds(w * 16 * ncand + t * 16,
                                                   16)]
                        d = d + jnp.where(tot <= kv, 1, 0)
                    newx = xb + d * q
                    if rnd < 15:
                        swbuf[...] = newx
                        pltpu.sync_copy(swbuf, st)
                    else:
                        swbuf[...] = newx ^ jnp.full((16,), -(2 ** 31),
                                                     jnp.int32)
                        pltpu.sync_copy(swbuf,
                                        out_hbm.at[pl.ds(gate * 16, 16)])

            plsc.subcore_barrier()

    return k(scores_biased, idx_rep)


def kernel(weight, steps):
    dens = []
    for k in range(3):
        r = 1.0 - (steps - _START) / (_END - _START)
        dens.append(jnp.where(steps < _END,
                              1.0 - (1.0 - _DENS[k]) * (1.0 - r ** 3),
                              _DENS[k]))
    nblk = _SR * _SC
    idx = jnp.stack([jnp.round(nblk * (1.0 - d)).astype(jnp.int32)
                     for d in dens])

    scores = pl.pallas_call(
        _scores_body,
        grid=(_NCHUNK,),
        in_specs=[pl.BlockSpec((_CHUNK, _NCOLS), lambda i: (i, 0))],
        out_specs=pl.BlockSpec((_SRC, _SC), lambda i: (i, 0)),
        out_shape=jax.ShapeDtypeStruct((3 * _SR, _SC), jnp.float32),
    )(weight)

    sbits = jax.lax.bitcast_convert_type(scores.reshape(-1), jnp.int32)
    thresh = _sc_select(sbits ^ jnp.int32(-(2 ** 31)), jnp.repeat(idx, 16))

    out = pl.pallas_call(
        _expand_body,
        grid=(_NCHUNK,),
        in_specs=[
            pl.BlockSpec(memory_space=pltpu.VMEM),
            pl.BlockSpec(memory_space=pltpu.SMEM),
        ],
        out_specs=pl.BlockSpec((_CHUNK, _NCOLS), lambda i: (i, 0)),
        out_shape=jax.ShapeDtypeStruct((_NROWS, _NCOLS), jnp.float32),
    )(scores, thresh)
    return out
